# baseline (speedup 1.0000x reference)
import functools

import jax
import jax.numpy as jnp
from jax import lax
from jax.experimental import pallas as pl
from jax.experimental.pallas import tpu as pltpu
from jax.experimental.pallas import tpu_sc as plsc

_B = 16384
_D = 64
_V = 1000000
_NC = 2
_NS = 16
_NW = _NC * _NS
_WINC = 512
_WPT = 62
_RANGE = _WPT * _WINC
_SELCAP = 640
_BINSTRIDE = _SELCAP + 16
_PSORT = 976
_FRROWS = 328
_FROVF = 24
_QSTEP = 80
_QROWS = 88


def _iota16():
    return lax.iota(jnp.int32, 16)


def _pop(mask):
    return plsc.all_reduce_population_count(mask)[0]


def _sc_body(fpad_hbm, lab_hbm, ct_hbm, tail_hbm, out_hbm,
             lab_v, sel_lab, sel_item, cb_lab, cb_item, ps_lab, ps_item,
             win0, win1, fr_v, part_v, smem_cb, smem_off, smem_cur,
             sem0, sem1, semf, semfo):
    wid = lax.axis_index("s") * _NC + lax.axis_index("c")
    lo = wid * _RANGE
    hi = lo + _RANGE
    nwin = jnp.where(wid < _NW - 1, _WPT, 32)

    part_v[...] = jnp.zeros((16,), jnp.float32)
    zeros16 = jnp.zeros((16,), jnp.int32)
    for z in range(_PSORT // 16):
        ps_item[pl.ds(z * 16, 16)] = zeros16

    def scan_pass(rank_lo):
        cnt0 = jnp.int32(0)
        gr0 = jnp.int32(0)
        for ch in range(4):
            pltpu.sync_copy(lab_hbm.at[pl.ds(ch * 4096, 4096)], lab_v)

            def sbody(v, carry, ch=ch):
                cnt, gr = carry
                l16 = lab_v[pl.ds(v * 16, 16)]
                m = (l16 >= lo) & (l16 < hi)
                incl = plsc.cumsum(
                    jnp.where(m, 1, 0).astype(jnp.int32), mask=m)
                lane_rank = gr + incl - 1
                m2 = m & (lane_rank >= rank_lo) & (
                    lane_rank < rank_lo + _SELCAP)
                plsc.store_compressed(sel_lab.at[pl.ds(cnt, 16)], l16,
                                      mask=m2)
                ids = _iota16() + (ch * 4096 + v * 16)
                plsc.store_compressed(sel_item.at[pl.ds(cnt, 16)], ids,
                                      mask=m2)
                return cnt + _pop(m2), gr + _pop(m)

            cnt0, gr0 = lax.fori_loop(0, 256, sbody, (cnt0, gr0))
        return cnt0, gr0

    def coarse_radix(nsel):
        def cbody(v, cnts):
            base = v * 16
            l16 = sel_lab[pl.ds(base, 16)]
            it16 = sel_item[pl.ds(base, 16)]
            valid = (base + _iota16()) < nsel
            cbid = lax.shift_right_logical(l16 - lo, 12)
            out = []
            for cbi in range(8):
                m = valid & (cbid == cbi)
                c = cnts[cbi]
                plsc.store_compressed(
                    cb_lab.at[pl.ds(cbi * _BINSTRIDE + c, 16)], l16, mask=m)
                plsc.store_compressed(
                    cb_item.at[pl.ds(cbi * _BINSTRIDE + c, 16)], it16, mask=m)
                out.append(c + _pop(m))
            return tuple(out)

        ntrip = lax.shift_right_logical(nsel + 15, 4)
        cnts = lax.fori_loop(0, ntrip, cbody,
                             tuple(jnp.int32(0) for _ in range(8)))
        for cbi in range(8):
            smem_cb[cbi] = cnts[cbi]

    def sub_radix_all(nsel):
        gbase = jnp.int32(0)
        for cb in range(8):
            n_cb = smem_cb[cb]
            off_cb = cb * _BINSTRIDE
            ntrip = lax.shift_right_logical(n_cb + 15, 4)

            def count_body(t, cnts, off_cb=off_cb, n_cb=n_cb):
                base = t * 16
                l16 = cb_lab[pl.ds(off_cb + base, 16)]
                valid = (base + _iota16()) < n_cb
                sbid = lax.shift_right_logical(l16 - lo, 9) & 7
                return tuple(cnts[s] + _pop(valid & (sbid == s))
                             for s in range(8))

            cnts = lax.fori_loop(0, ntrip, count_body,
                                 tuple(jnp.int32(0) for _ in range(8)))
            offs = []
            run = gbase
            for s in range(8):
                smem_off[cb * 8 + s] = run
                offs.append(run)
                run = run + cnts[s]
            gbase = run

            def place_body(t, curs, off_cb=off_cb, n_cb=n_cb):
                base = t * 16
                l16 = cb_lab[pl.ds(off_cb + base, 16)]
                it16 = cb_item[pl.ds(off_cb + base, 16)]
                valid = (base + _iota16()) < n_cb
                sbid = lax.shift_right_logical(l16 - lo, 9) & 7
                out = []
                for s in range(8):
                    m = valid & (sbid == s)
                    c = curs[s]
                    plsc.store_compressed(ps_lab.at[pl.ds(c, 16)], l16,
                                          mask=m)
                    plsc.store_compressed(ps_item.at[pl.ds(c, 16)], it16,
                                          mask=m)
                    out.append(c + _pop(m))
                return tuple(out)

            lax.fori_loop(0, ntrip, place_body, tuple(offs))
        ps_item[pl.ds(gbase, 16)] = zeros16
        smem_off[_WPT] = gbase

    def win_off(k):
        return pl.multiple_of((wid * _WPT + k) * _WINC, _WINC)

    def dma_ok(k):
        return (k < _WPT) & ((wid < _NW - 1) | (k < 31))

    def fire_win(k, buf, sem):
        @pl.when(dma_ok(k))
        def _():
            pltpu.async_copy(ct_hbm.at[:, pl.ds(win_off(k), _WINC)],
                             buf, sem)

    def fire_half(h):
        ab = pl.multiple_of(jnp.where(h == 0, 0, smem_off[32] & ~7), 8)
        for q in range(4):
            pltpu.async_copy(
                fpad_hbm.at[ps_item.at[pl.ds(ab + q * _QSTEP, _QROWS)]],
                fr_v.at[pl.ds(q * _QSTEP, _QROWS)], semf)
        smem_cur[0] = 0

    def wait_quarter(q):
        pltpu.make_async_copy(
            fpad_hbm.at[ps_item.at[pl.ds(0, _QROWS)]],
            fr_v.at[pl.ds(0, _QROWS)], semf).wait()

    def drain_half():
        cur = smem_cur[0]
        for q in range(4):
            @pl.when(cur <= q)
            def _():
                wait_quarter(q)
        smem_cur[0] = 4

    def process(k, buf):
        off = smem_off[k]
        n = smem_off[k + 1] - off
        off32a = smem_off[32] & ~7
        habase = jnp.where(k < 32, 0, off32a)
        hend = jnp.where(k < 32, smem_off[32], smem_off[_WPT])
        fb = (hend - habase) > _FRROWS
        win_base = lo + k * _WINC

        cur = smem_cur[0]
        rel_end = jnp.minimum(smem_off[k + 1] - habase, _FRROWS)
        needed = jnp.where(fb, 3, jnp.minimum((rel_end + 14) // _QSTEP, 3))
        for q in range(4):
            @pl.when((cur <= q) & (q <= needed))
            def _():
                wait_quarter(q)
        smem_cur[0] = jnp.maximum(cur, needed + 1)

        def gbody(t, _):
            row0 = t * 16

            @pl.when(fb)
            def _():
                ab = pl.multiple_of((off + row0) & ~7, 8)
                pltpu.async_copy(
                    fpad_hbm.at[ps_item.at[pl.ds(ab, _FROVF)]],
                    fr_v.at[pl.ds(0, _FROVF)], semfo).wait()

            rel = jnp.where(fb, (off + row0) & 7, (off + row0) - habase)
            l16 = ps_lab[pl.ds(off + row0, 16)]
            gmask = (row0 + _iota16()) < n
            widx = (l16 - win_base) & (_WINC - 1)
            maskf = jnp.where(gmask, 1.0, 0.0).astype(jnp.float32)
            acc = jnp.zeros((16,), jnp.float32)
            rows = jnp.minimum(rel + _iota16(), _FRROWS - 1)
            for c in range(_D):
                fv = plsc.load_gather(
                    fr_v, [rows, jnp.full((16,), c, jnp.int32)])
                cv = plsc.load_gather(
                    buf, [jnp.full((16,), c, jnp.int32), widx])
                d = fv - cv
                acc = acc + d * d * maskf
            part_v[...] = part_v[...] + acc
            return 0

        ngroups = lax.shift_right_logical(n + 15, 4)
        lax.fori_loop(0, ngroups, gbody, 0)

    def run_windows():
        fire_win(0, win0, sem0)
        fire_win(1, win1, sem1)
        fire_half(0)

        def wbody(g, _):
            for b, (buf, sem) in enumerate(((win0, sem0), (win1, sem1))):
                k = 2 * g + b

                @pl.when(dma_ok(k))
                def _(buf=buf, sem=sem, k=k):
                    pltpu.make_async_copy(
                        ct_hbm.at[:, pl.ds(win_off(k), _WINC)], buf,
                        sem).wait()

                @pl.when((wid == _NW - 1) & (k == 31))
                def _(buf=buf):
                    pltpu.sync_copy(tail_hbm, buf.at[:, pl.ds(0, 128)])

                @pl.when(k < nwin)
                def _(buf=buf, k=k):
                    process(k, buf)

                fire_win(k + 2, buf, sem)

                @pl.when(k == 31)
                def _():
                    drain_half()
                    fire_half(1)
            return 0

        lax.fori_loop(0, _WPT // 2, wbody, 0)
        drain_half()

    def pass_body(state):
        rank_lo, _ = state
        nsel, total = scan_pass(rank_lo)
        coarse_radix(nsel)
        sub_radix_all(nsel)
        run_windows()
        return rank_lo + _SELCAP, total

    def pass_cond(state):
        rank_lo, total = state
        return rank_lo < total

    lax.while_loop(pass_cond, pass_body, (jnp.int32(0), jnp.int32(1)))

    pltpu.sync_copy(part_v, out_hbm.at[wid])


@jax.jit
def _center_loss_sc(fpad, labels, centers_t, tail):
    mesh = plsc.VectorSubcoreMesh(core_axis_name="c", subcore_axis_name="s",
                                  num_cores=_NC, num_subcores=_NS)
    k = pl.kernel(
        _sc_body,
        out_type=jax.ShapeDtypeStruct((_NW, 16), jnp.float32),
        mesh=mesh,
        scratch_types=[
            pltpu.VMEM((4096,), jnp.int32),
            pltpu.VMEM((_BINSTRIDE,), jnp.int32),
            pltpu.VMEM((_BINSTRIDE,), jnp.int32),
            pltpu.VMEM((8 * _BINSTRIDE,), jnp.int32),
            pltpu.VMEM((8 * _BINSTRIDE,), jnp.int32),
            pltpu.VMEM((_PSORT,), jnp.int32),
            pltpu.VMEM((_PSORT,), jnp.int32),
            pltpu.VMEM((_D, _WINC), jnp.float32),
            pltpu.VMEM((_D, _WINC), jnp.float32),
            pltpu.VMEM((_FRROWS, 128), jnp.float32),
            pltpu.VMEM((16,), jnp.float32),
            pltpu.SMEM((8,), jnp.int32),
            pltpu.SMEM((_WPT + 2,), jnp.int32),
            pltpu.SMEM((4,), jnp.int32),
            pltpu.SemaphoreType.DMA,
            pltpu.SemaphoreType.DMA,
            pltpu.SemaphoreType.DMA,
            pltpu.SemaphoreType.DMA,
        ],
        compiler_params=pltpu.CompilerParams(needs_layout_passes=False),
    )
    return k(fpad, labels, centers_t, tail)


def kernel(features, labels, centers):
    fpad = jnp.pad(features, ((0, 0), (0, 128 - _D)))
    centers_t = centers.T
    tail = jnp.pad(centers_t[:, _V - 64:], ((0, 0), (0, 64)))
    parts = _center_loss_sc(fpad, labels.astype(jnp.int32), centers_t, tail)
    return jnp.sum(parts) / features.shape[0]


# speedup vs baseline: 1.0333x; 1.0333x over previous
import functools

import jax
import jax.numpy as jnp
from jax import lax
from jax.experimental import pallas as pl
from jax.experimental.pallas import tpu as pltpu
from jax.experimental.pallas import tpu_sc as plsc

_B = 16384
_D = 64
_V = 1000000
_NC = 2
_NS = 16
_NW = _NC * _NS
_WINC = 512
_WPT = 62
_RANGE = _WPT * _WINC
_SELCAP = 640
_BINSTRIDE = _SELCAP + 16
_PSORT = 976
_FRROWS = 328
_FROVF = 24


def _iota16():
    return lax.iota(jnp.int32, 16)


def _pop(mask):
    return plsc.all_reduce_population_count(mask)[0]


def _sc_body(fpad_hbm, lab_hbm, ct_hbm, tail_hbm, out_hbm,
             lab_v, sel_lab, sel_item, cb_lab, cb_item, ps_lab, ps_item,
             win0, win1, fr_v, part_v, smem_cb, smem_off, smem_cur,
             sem0, sem1, semf, semfo):
    wid = lax.axis_index("s") * _NC + lax.axis_index("c")
    lo = wid * _RANGE
    hi = lo + _RANGE
    nwin = jnp.where(wid < _NW - 1, _WPT, 32)

    part_v[...] = jnp.zeros((16,), jnp.float32)
    zeros16 = jnp.zeros((16,), jnp.int32)
    for z in range(_PSORT // 16):
        ps_item[pl.ds(z * 16, 16)] = zeros16

    def scan_pass(rank_lo):
        cnt0 = jnp.int32(0)
        gr0 = jnp.int32(0)
        for ch in range(4):
            pltpu.sync_copy(lab_hbm.at[pl.ds(ch * 4096, 4096)], lab_v)

            def sbody(v, carry, ch=ch):
                cnt, gr = carry
                l16 = lab_v[pl.ds(v * 16, 16)]
                m = (l16 >= lo) & (l16 < hi)
                incl = plsc.cumsum(
                    jnp.where(m, 1, 0).astype(jnp.int32), mask=m)
                lane_rank = gr + incl - 1
                m2 = m & (lane_rank >= rank_lo) & (
                    lane_rank < rank_lo + _SELCAP)
                plsc.store_compressed(sel_lab.at[pl.ds(cnt, 16)], l16,
                                      mask=m2)
                ids = _iota16() + (ch * 4096 + v * 16)
                plsc.store_compressed(sel_item.at[pl.ds(cnt, 16)], ids,
                                      mask=m2)
                return cnt + _pop(m2), gr + _pop(m)

            cnt0, gr0 = lax.fori_loop(0, 256, sbody, (cnt0, gr0))
        return cnt0, gr0

    def coarse_radix(nsel):
        def cbody(v, cnts):
            base = v * 16
            l16 = sel_lab[pl.ds(base, 16)]
            it16 = sel_item[pl.ds(base, 16)]
            valid = (base + _iota16()) < nsel
            cbid = lax.shift_right_logical(l16 - lo, 12)
            out = []
            for cbi in range(8):
                m = valid & (cbid == cbi)
                c = cnts[cbi]
                plsc.store_compressed(
                    cb_lab.at[pl.ds(cbi * _BINSTRIDE + c, 16)], l16, mask=m)
                plsc.store_compressed(
                    cb_item.at[pl.ds(cbi * _BINSTRIDE + c, 16)], it16, mask=m)
                out.append(c + _pop(m))
            return tuple(out)

        ntrip = lax.shift_right_logical(nsel + 15, 4)
        cnts = lax.fori_loop(0, ntrip, cbody,
                             tuple(jnp.int32(0) for _ in range(8)))
        for cbi in range(8):
            smem_cb[cbi] = cnts[cbi]

    def sub_radix_all(nsel):
        gbase = jnp.int32(0)
        for cb in range(8):
            n_cb = smem_cb[cb]
            off_cb = cb * _BINSTRIDE
            ntrip = lax.shift_right_logical(n_cb + 15, 4)

            def count_body(t, cnts, off_cb=off_cb, n_cb=n_cb):
                base = t * 16
                l16 = cb_lab[pl.ds(off_cb + base, 16)]
                valid = (base + _iota16()) < n_cb
                sbid = lax.shift_right_logical(l16 - lo, 9) & 7
                return tuple(cnts[s] + _pop(valid & (sbid == s))
                             for s in range(8))

            cnts = lax.fori_loop(0, ntrip, count_body,
                                 tuple(jnp.int32(0) for _ in range(8)))
            offs = []
            run = gbase
            for s in range(8):
                smem_off[cb * 8 + s] = run
                offs.append(run)
                run = run + cnts[s]
            gbase = run

            def place_body(t, curs, off_cb=off_cb, n_cb=n_cb):
                base = t * 16
                l16 = cb_lab[pl.ds(off_cb + base, 16)]
                it16 = cb_item[pl.ds(off_cb + base, 16)]
                valid = (base + _iota16()) < n_cb
                sbid = lax.shift_right_logical(l16 - lo, 9) & 7
                out = []
                for s in range(8):
                    m = valid & (sbid == s)
                    c = curs[s]
                    plsc.store_compressed(ps_lab.at[pl.ds(c, 16)], l16,
                                          mask=m)
                    plsc.store_compressed(ps_item.at[pl.ds(c, 16)], it16,
                                          mask=m)
                    out.append(c + _pop(m))
                return tuple(out)

            lax.fori_loop(0, ntrip, place_body, tuple(offs))
        ps_item[pl.ds(gbase, 16)] = zeros16
        smem_off[_WPT] = gbase

    def win_off(k):
        return pl.multiple_of((wid * _WPT + k) * _WINC, _WINC)

    def dma_ok(k):
        return (k < _WPT) & ((wid < _NW - 1) | (k < 31))

    def fire_win(k, buf, sem):
        @pl.when(dma_ok(k))
        def _():
            pltpu.async_copy(ct_hbm.at[:, pl.ds(win_off(k), _WINC)],
                             buf, sem)

    def fire_half(h):
        ab = pl.multiple_of(jnp.where(h == 0, 0, smem_off[32] & ~7), 8)
        pltpu.async_copy(
            fpad_hbm.at[ps_item.at[pl.ds(ab, _FRROWS)]], fr_v, semf)
        smem_cur[0] = 0

    def drain_half():
        @pl.when(smem_cur[0] == 0)
        def _():
            pltpu.make_async_copy(
                fpad_hbm.at[ps_item.at[pl.ds(0, _FRROWS)]], fr_v,
                semf).wait()
        smem_cur[0] = 1

    def process(k, buf):
        off = smem_off[k]
        n = smem_off[k + 1] - off
        off32a = smem_off[32] & ~7
        habase = jnp.where(k < 32, 0, off32a)
        hend = jnp.where(k < 32, smem_off[32], smem_off[_WPT])
        fb = (hend - habase) > _FRROWS
        win_base = lo + k * _WINC

        @pl.when((smem_cur[0] == 0) & (n > 0))
        def _():
            pltpu.make_async_copy(
                fpad_hbm.at[ps_item.at[pl.ds(0, _FRROWS)]], fr_v,
                semf).wait()
            smem_cur[0] = 1

        def gbody(t, _):
            row0 = t * 16

            @pl.when(fb)
            def _():
                ab = pl.multiple_of((off + row0) & ~7, 8)
                pltpu.async_copy(
                    fpad_hbm.at[ps_item.at[pl.ds(ab, _FROVF)]],
                    fr_v.at[pl.ds(0, _FROVF)], semfo).wait()

            rel = jnp.where(fb, (off + row0) & 7, (off + row0) - habase)
            l16 = ps_lab[pl.ds(off + row0, 16)]
            gmask = (row0 + _iota16()) < n
            widx = (l16 - win_base) & (_WINC - 1)
            maskf = jnp.where(gmask, 1.0, 0.0).astype(jnp.float32)
            acc = jnp.zeros((16,), jnp.float32)
            rows = jnp.minimum(rel + _iota16(), _FRROWS - 1)
            for c in range(_D):
                fv = plsc.load_gather(
                    fr_v, [rows, jnp.full((16,), c, jnp.int32)])
                cv = plsc.load_gather(
                    buf, [jnp.full((16,), c, jnp.int32), widx])
                d = fv - cv
                acc = acc + d * d * maskf
            part_v[...] = part_v[...] + acc
            return 0

        ngroups = lax.shift_right_logical(n + 15, 4)
        lax.fori_loop(0, ngroups, gbody, 0)

    def run_windows():
        fire_win(0, win0, sem0)
        fire_win(1, win1, sem1)
        fire_half(0)

        def wbody(g, _):
            for b, (buf, sem) in enumerate(((win0, sem0), (win1, sem1))):
                k = 2 * g + b

                @pl.when(dma_ok(k))
                def _(buf=buf, sem=sem, k=k):
                    pltpu.make_async_copy(
                        ct_hbm.at[:, pl.ds(win_off(k), _WINC)], buf,
                        sem).wait()

                @pl.when((wid == _NW - 1) & (k == 31))
                def _(buf=buf):
                    pltpu.sync_copy(tail_hbm, buf.at[:, pl.ds(0, 128)])

                @pl.when(k < nwin)
                def _(buf=buf, k=k):
                    process(k, buf)

                fire_win(k + 2, buf, sem)

                @pl.when(k == 31)
                def _():
                    drain_half()
                    fire_half(1)
            return 0

        lax.fori_loop(0, _WPT // 2, wbody, 0)
        drain_half()

    def pass_body(state):
        rank_lo, _ = state
        nsel, total = scan_pass(rank_lo)
        coarse_radix(nsel)
        sub_radix_all(nsel)
        run_windows()
        return rank_lo + _SELCAP, total

    def pass_cond(state):
        rank_lo, total = state
        return rank_lo < total

    lax.while_loop(pass_cond, pass_body, (jnp.int32(0), jnp.int32(1)))

    pltpu.sync_copy(part_v, out_hbm.at[wid])


@jax.jit
def _center_loss_sc(fpad, labels, centers_t, tail):
    mesh = plsc.VectorSubcoreMesh(core_axis_name="c", subcore_axis_name="s",
                                  num_cores=_NC, num_subcores=_NS)
    k = pl.kernel(
        _sc_body,
        out_type=jax.ShapeDtypeStruct((_NW, 16), jnp.float32),
        mesh=mesh,
        scratch_types=[
            pltpu.VMEM((4096,), jnp.int32),
            pltpu.VMEM((_BINSTRIDE,), jnp.int32),
            pltpu.VMEM((_BINSTRIDE,), jnp.int32),
            pltpu.VMEM((8 * _BINSTRIDE,), jnp.int32),
            pltpu.VMEM((8 * _BINSTRIDE,), jnp.int32),
            pltpu.VMEM((_PSORT,), jnp.int32),
            pltpu.VMEM((_PSORT,), jnp.int32),
            pltpu.VMEM((_D, _WINC), jnp.float32),
            pltpu.VMEM((_D, _WINC), jnp.float32),
            pltpu.VMEM((_FRROWS, 128), jnp.float32),
            pltpu.VMEM((16,), jnp.float32),
            pltpu.SMEM((8,), jnp.int32),
            pltpu.SMEM((_WPT + 2,), jnp.int32),
            pltpu.SMEM((4,), jnp.int32),
            pltpu.SemaphoreType.DMA,
            pltpu.SemaphoreType.DMA,
            pltpu.SemaphoreType.DMA,
            pltpu.SemaphoreType.DMA,
        ],
        compiler_params=pltpu.CompilerParams(needs_layout_passes=False),
    )
    return k(fpad, labels, centers_t, tail)


def kernel(features, labels, centers):
    fpad = jnp.pad(features, ((0, 0), (0, 128 - _D)))
    centers_t = centers.T
    tail = jnp.pad(centers_t[:, _V - 64:], ((0, 0), (0, 64)))
    parts = _center_loss_sc(fpad, labels.astype(jnp.int32), centers_t, tail)
    return jnp.sum(parts) / features.shape[0]
